# baseline (speedup 1.0000x reference)
import jax
import jax.numpy as jnp
from jax import lax
from jax.experimental import pallas as pl
from jax.experimental.pallas import tpu as pltpu
from jax.experimental.pallas import tpu_sc as plsc

B = 16384
D = 50
SPAN = 4
SR = D * SPAN
SRN = 100000 // SPAN
L = 16
NC = 2
NS = 16
NW = NC * NS
BPW = B // NW
CH = 128
NCH = BPW // CH


def _rsqrt16(x):
    i = plsc.bitcast(x, jnp.int32)
    i = jnp.int32(0x5F3759DF) - lax.shift_right_logical(i, 1)
    y = plsc.bitcast(i, jnp.float32)
    half = x * 0.5
    for _ in range(3):
        y = y * (1.5 - half * y * y)
    return y


def _sc_body(fg_hbm, tg_hbm, fm_hbm, tm_hbm, ftab_hbm, ttab_hbm, wb_hbm,
             out_hbm,
             fg_v, tg_v, fm_v, tm_v, fbufA, tbufA, fbufB, tbufB, out_v,
             wb_v, semA, semB):
    wid = lax.axis_index("s") * NC + lax.axis_index("c")
    base = wid * BPW

    pltpu.sync_copy(fg_hbm.at[wid], fg_v)
    pltpu.sync_copy(tg_hbm.at[wid], tg_v)
    pltpu.sync_copy(fm_hbm.at[wid], fm_v)
    pltpu.sync_copy(tm_hbm.at[wid], tm_v)
    pltpu.sync_copy(wb_hbm, wb_v)

    w = wb_v[pl.ds(0, L)]
    bb = wb_v[pl.ds(L, L)]
    lanes = lax.iota(jnp.int32, L)
    bufs = ((fbufA, tbufA, semA), (fbufB, tbufB, semB))

    def fire(c):
        fb, tb, sm = bufs[c % 2]
        return (pltpu.async_copy(ftab_hbm.at[fg_v.at[c]], fb, sm),
                pltpu.async_copy(ttab_hbm.at[tg_v.at[c]], tb, sm))

    pend = fire(0)
    for c in range(NCH):
        nxt = fire(c + 1) if c + 1 < NCH else None
        for d in pend:
            d.wait()
        fb, tb, _ = bufs[c % 2]

        def block(b, carry, fb=fb, tb=tb, c=c):
            row16 = lanes + b * L
            g = c * CH + b * L
            fmv = fm_v[pl.ds(g, L)]
            tmv = tm_v[pl.ds(g, L)]
            acc_d = jnp.zeros((L,), jnp.float32)
            acc_f = jnp.zeros((L,), jnp.float32)
            acc_t = jnp.zeros((L,), jnp.float32)
            for e in range(D):
                fe = plsc.load_gather(fb, [row16, fmv + e])
                te = plsc.load_gather(tb, [row16, tmv + e])
                acc_d = acc_d + fe * te
                acc_f = acc_f + fe * fe
                acc_t = acc_t + te * te
            rs = (_rsqrt16(jnp.maximum(acc_f, 1e-12))
                  * _rsqrt16(jnp.maximum(acc_t, 1e-12)))
            z = acc_d * rs * w + bb
            res = 1.0 / (1.0 + jnp.exp(-z))
            out_v[pl.ds(g, L)] = res
            return carry

        lax.fori_loop(0, CH // L, block, 0)
        pend = nxt

    pltpu.sync_copy(out_v, out_hbm.at[pl.ds(base, BPW)])


@jax.jit
def _sc_call(fg, tg, fm, tm, ftab, ttab, wb):
    mesh = plsc.VectorSubcoreMesh(
        core_axis_name="c", subcore_axis_name="s",
        num_cores=NC, num_subcores=NS)
    return pl.kernel(
        _sc_body,
        out_type=jax.ShapeDtypeStruct((B,), jnp.float32),
        mesh=mesh,
        compiler_params=pltpu.CompilerParams(
            needs_layout_passes=False, use_tc_tiling_on_sc=False),
        scratch_types=[
            pltpu.VMEM((NCH, CH), jnp.int32),
            pltpu.VMEM((NCH, CH), jnp.int32),
            pltpu.VMEM((BPW,), jnp.int32),
            pltpu.VMEM((BPW,), jnp.int32),
            pltpu.VMEM((CH, SR), jnp.float32),
            pltpu.VMEM((CH, SR), jnp.float32),
            pltpu.VMEM((CH, SR), jnp.float32),
            pltpu.VMEM((CH, SR), jnp.float32),
            pltpu.VMEM((BPW,), jnp.float32),
            pltpu.VMEM((2 * L,), jnp.float32),
            pltpu.SemaphoreType.DMA,
            pltpu.SemaphoreType.DMA,
        ],
    )(fg, tg, fm, tm, ftab, ttab, wb)


def kernel(file, test, file_table, test_table, W, b):
    fidx = file.reshape(B)
    tidx = test.reshape(B)
    fg = (fidx // SPAN).reshape(NW, NCH, CH)
    tg = (tidx // SPAN).reshape(NW, NCH, CH)
    fm = ((fidx % SPAN) * D).reshape(NW, BPW)
    tm = ((tidx % SPAN) * D).reshape(NW, BPW)
    ftab = file_table.reshape(SRN, SR)
    ttab = test_table.reshape(SRN, SR)
    wb = jnp.concatenate([
        jnp.broadcast_to(W.reshape(1), (L,)),
        jnp.broadcast_to(b.reshape(1), (L,)),
    ]).astype(jnp.float32)
    out = _sc_call(fg, tg, fm, tm, ftab, ttab, wb)
    return out.reshape(B, 1)


# speedup vs baseline: 1.5609x; 1.5609x over previous
import jax
import jax.numpy as jnp
from jax import lax
from jax.experimental import pallas as pl
from jax.experimental.pallas import tpu as pltpu
from jax.experimental.pallas import tpu_sc as plsc

B = 16384
D = 50
L = 16
NC = 2
NS = 16
NW = NC * NS
BPW = B // NW
CH = 128
NCH = BPW // CH


def _rsqrt16(x):
    i = plsc.bitcast(x, jnp.int32)
    i = jnp.int32(0x5F3759DF) - lax.shift_right_logical(i, 1)
    y = plsc.bitcast(i, jnp.float32)
    half = x * 0.5
    for _ in range(3):
        y = y * (1.5 - half * y * y)
    return y


def _sc_body(fg_hbm, tg_hbm, ftab_hbm, ttab_hbm, wb_hbm,
             out_hbm,
             fg_v, tg_v, fbufA, tbufA, fbufB, tbufB, out_v,
             wb_v, semA, semB):
    wid = lax.axis_index("s") * NC + lax.axis_index("c")
    base = wid * BPW

    pltpu.sync_copy(fg_hbm.at[wid], fg_v)
    pltpu.sync_copy(tg_hbm.at[wid], tg_v)
    pltpu.sync_copy(wb_hbm, wb_v)

    w = wb_v[pl.ds(0, L)]
    bb = wb_v[pl.ds(L, L)]
    lanes = lax.iota(jnp.int32, L)
    bufs = ((fbufA, tbufA, semA), (fbufB, tbufB, semB))

    def fire(c):
        fb, tb, sm = bufs[c % 2]

        def issue(b, carry):
            fi16 = fg_v[c, pl.ds(b * L, L)]
            ti16 = tg_v[c, pl.ds(b * L, L)]
            for j in range(L):
                pltpu.async_copy(ftab_hbm.at[fi16[j]], fb.at[b * L + j], sm)
                pltpu.async_copy(ttab_hbm.at[ti16[j]], tb.at[b * L + j], sm)
            return carry

        lax.fori_loop(0, CH // L, issue, 0)

    def drain(c):
        fb, tb, sm = bufs[c % 2]

        def wait1(i, carry):
            pltpu.make_async_copy(ftab_hbm.at[0], fb.at[0], sm).wait()
            pltpu.make_async_copy(ttab_hbm.at[0], tb.at[0], sm).wait()
            return carry

        lax.fori_loop(0, CH, wait1, 0)

    fire(0)
    for c in range(NCH):
        if c + 1 < NCH:
            fire(c + 1)
        drain(c)
        fb, tb, _ = bufs[c % 2]

        def block(b, carry, fb=fb, tb=tb, c=c):
            row16 = lanes + b * L
            g = c * CH + b * L
            acc_d = jnp.zeros((L,), jnp.float32)
            acc_f = jnp.zeros((L,), jnp.float32)
            acc_t = jnp.zeros((L,), jnp.float32)
            for e in range(D):
                col = jnp.full((L,), e, jnp.int32)
                fe = plsc.load_gather(fb, [row16, col])
                te = plsc.load_gather(tb, [row16, col])
                acc_d = acc_d + fe * te
                acc_f = acc_f + fe * fe
                acc_t = acc_t + te * te
            rs = (_rsqrt16(jnp.maximum(acc_f, 1e-12))
                  * _rsqrt16(jnp.maximum(acc_t, 1e-12)))
            z = acc_d * rs * w + bb
            res = 1.0 / (1.0 + jnp.exp(-z))
            out_v[pl.ds(g, L)] = res
            return carry

        lax.fori_loop(0, CH // L, block, 0)

    pltpu.sync_copy(out_v, out_hbm.at[pl.ds(base, BPW)])


@jax.jit
def _sc_call(fg, tg, ftab, ttab, wb):
    mesh = plsc.VectorSubcoreMesh(
        core_axis_name="c", subcore_axis_name="s",
        num_cores=NC, num_subcores=NS)
    return pl.kernel(
        _sc_body,
        out_type=jax.ShapeDtypeStruct((B,), jnp.float32),
        mesh=mesh,
        compiler_params=pltpu.CompilerParams(
            needs_layout_passes=False),
        scratch_types=[
            pltpu.VMEM((NCH, CH), jnp.int32),
            pltpu.VMEM((NCH, CH), jnp.int32),
            pltpu.VMEM((CH, D), jnp.float32),
            pltpu.VMEM((CH, D), jnp.float32),
            pltpu.VMEM((CH, D), jnp.float32),
            pltpu.VMEM((CH, D), jnp.float32),
            pltpu.VMEM((BPW,), jnp.float32),
            pltpu.VMEM((2 * L,), jnp.float32),
            pltpu.SemaphoreType.DMA,
            pltpu.SemaphoreType.DMA,
        ],
    )(fg, tg, ftab, ttab, wb)


def kernel(file, test, file_table, test_table, W, b):
    fidx = file.reshape(B)
    tidx = test.reshape(B)
    fg = fidx.reshape(NW, NCH, CH)
    tg = tidx.reshape(NW, NCH, CH)
    wb = jnp.concatenate([
        jnp.broadcast_to(W.reshape(1), (L,)),
        jnp.broadcast_to(b.reshape(1), (L,)),
    ]).astype(jnp.float32)
    out = _sc_call(fg, tg, file_table, test_table, wb)
    return out.reshape(B, 1)
